# Initial kernel scaffold; baseline (speedup 1.0000x reference)
import jax
import jax.numpy as jnp
from jax.experimental import pallas as pl


def kernel(cat_id, amount_bucket, channel_id, days_delta, cat_table, bucket_table, channel_table, days_w, days_b, proj_w, proj_b, ln_g, ln_b):
    raise NotImplementedError("write your pallas kernel here")


# speedup vs baseline: 5.2180x; 5.2180x over previous
import jax
import jax.numpy as jnp
from jax.experimental import pallas as pl

MAX_DAYS = 365.0
TM = 8192


def _body(cat_ref, bkt_ref, chn_ref, days_ref,
          wemb_ref, wdays_ref, dw_ref, db_ref, pb_ref, g_ref, b_ref,
          out_ref):
    iota = jax.lax.broadcasted_iota(jnp.int32, (TM, 32), 1)
    oh = ((cat_ref[:] == iota).astype(jnp.float32)
          + (bkt_ref[:] + 13 == iota).astype(jnp.float32)
          + (chn_ref[:] + 19 == iota).astype(jnp.float32))
    dfeat = jnp.maximum(days_ref[:] * dw_ref[:] + db_ref[:], 0.0)
    h = (jnp.dot(oh, wemb_ref[:], preferred_element_type=jnp.float32)
         + jnp.dot(dfeat, wdays_ref[:], preferred_element_type=jnp.float32)
         + pb_ref[:])
    mu = jnp.mean(h, axis=1, keepdims=True)
    c = h - mu
    var = jnp.mean(c * c, axis=1, keepdims=True)
    out_ref[:] = c * jax.lax.rsqrt(var + 1e-5) * g_ref[:] + b_ref[:]


def kernel(cat_id, amount_bucket, channel_id, days_delta,
           cat_table, bucket_table, channel_table,
           days_w, days_b, proj_w, proj_b, ln_g, ln_b):
    B, S = cat_id.shape
    M = B * S
    D = proj_w.shape[0]

    catP = cat_table @ proj_w[:, 0:16].T
    bucketP = bucket_table @ proj_w[:, 16:24].T
    chanP = channel_table @ proj_w[:, 40:48].T
    wemb = jnp.zeros((32, D), jnp.float32)
    wemb = wemb.at[0:13].set(catP).at[13:19].set(bucketP).at[19:27].set(chanP)
    wdays = proj_w[:, 24:40].T
    dw = (days_w / MAX_DAYS).reshape(1, -1)
    db = days_b.reshape(1, -1)
    pb = proj_b.reshape(1, -1)
    g = ln_g.reshape(1, -1)
    b = ln_b.reshape(1, -1)

    col = lambda x: x.reshape(M, 1)
    grid = M // TM
    full = lambda shape: pl.BlockSpec(shape, lambda i: (0, 0))
    tok = pl.BlockSpec((TM, 1), lambda i: (i, 0))

    out = pl.pallas_call(
        _body,
        grid=(grid,),
        in_specs=[tok, tok, tok, tok,
                  full((32, D)), full((16, D)), full((1, 16)), full((1, 16)),
                  full((1, D)), full((1, D)), full((1, D))],
        out_specs=pl.BlockSpec((TM, D), lambda i: (i, 0)),
        out_shape=jax.ShapeDtypeStruct((M, D), jnp.float32),
    )(col(cat_id), col(amount_bucket), col(channel_id), col(days_delta),
      wemb, wdays, dw, db, pb, g, b)
    return out.reshape(B, S, D)
